# baseline (speedup 1.0000x reference)
import functools

import jax
import jax.numpy as jnp
from jax import lax
from jax.experimental import pallas as pl
from jax.experimental.pallas import tpu as pltpu
from jax.experimental.pallas import tpu_sc as plsc

T, D, H, E = 4096, 768, 3072, 8
CAP = 1280
DUMP = E * CAP
DISP_ROWS = E * CAP + 8

_CH = 512
_NCH = T // _CH


def _gating_body(x_ref, wg_ref, a1s_ref, a2s_ref, a1c_ref, a2c_ref,
                 g1_ref, g2_ref):
    x = x_ref[...]
    wg = wg_ref[...]
    logits = jnp.dot(x, wg, preferred_element_type=jnp.float32)
    m = jnp.max(logits, axis=1, keepdims=True)
    ex = jnp.exp(logits - m)
    s = ex / jnp.sum(ex, axis=1, keepdims=True)

    lane = lax.broadcasted_iota(jnp.int32, (T, E), 1)
    v1 = jnp.max(s, axis=1, keepdims=True)
    i1 = jnp.min(jnp.where(s == v1, lane, E), axis=1, keepdims=True)
    s2 = jnp.where(lane == i1, -jnp.inf, s)
    v2 = jnp.max(s2, axis=1, keepdims=True)
    i2 = jnp.min(jnp.where(s2 == v2, lane, E), axis=1, keepdims=True)

    denom = v1 + v2 + 1e-9
    g1 = v1 / denom
    g2 = v2 / denom

    mask1 = (lane == i1).astype(jnp.float32)
    mask2 = (lane == i2).astype(jnp.float32)
    c1tot = jnp.sum(mask1, axis=0, keepdims=True)

    ri = lax.broadcasted_iota(jnp.int32, (_CH, _CH), 0)
    ci = lax.broadcasted_iota(jnp.int32, (_CH, _CH), 1)
    tril = (ci < ri).astype(jnp.float32)

    c1run = jnp.zeros((1, E), jnp.float32)
    c2run = jnp.zeros((1, E), jnp.float32)
    for j in range(_NCH):
        sl = slice(j * _CH, (j + 1) * _CH)
        m1 = mask1[sl]
        m2 = mask2[sl]
        loc1 = jnp.dot(tril, m1, preferred_element_type=jnp.float32) + c1run
        loc2 = (jnp.dot(tril, m2, preferred_element_type=jnp.float32)
                + c2run + c1tot)
        pos1 = jnp.sum(loc1 * m1, axis=1, keepdims=True)
        pos2 = jnp.sum(loc2 * m2, axis=1, keepdims=True)
        keep1 = pos1 < CAP
        keep2 = pos2 < CAP
        p1 = jnp.minimum(pos1, CAP - 1).astype(jnp.int32)
        p2 = jnp.minimum(pos2, CAP - 1).astype(jnp.int32)
        addr1 = i1[sl] * CAP + p1
        addr2 = i2[sl] * CAP + p2
        a1s_ref[sl, :] = jnp.where(keep1, addr1, DUMP)
        a2s_ref[sl, :] = jnp.where(keep2, addr2, DUMP)
        a1c_ref[sl, :] = addr1
        a2c_ref[sl, :] = addr2
        g1_ref[sl, :] = jnp.where(keep1, g1[sl], 0.0)
        g2_ref[sl, :] = jnp.where(keep2, g2[sl], 0.0)
        c1run = c1run + jnp.sum(m1, axis=0, keepdims=True)
        c2run = c2run + jnp.sum(m2, axis=0, keepdims=True)


def _gating(x, wg):
    i32 = jnp.int32
    f32 = jnp.float32
    return pl.pallas_call(
        _gating_body,
        out_shape=[
            jax.ShapeDtypeStruct((T, 1), i32),
            jax.ShapeDtypeStruct((T, 1), i32),
            jax.ShapeDtypeStruct((T, 1), i32),
            jax.ShapeDtypeStruct((T, 1), i32),
            jax.ShapeDtypeStruct((T, 1), f32),
            jax.ShapeDtypeStruct((T, 1), f32),
        ],
    )(x, wg)


_NW = 32
_TPW = 2 * T // _NW
_SUB = 128


@functools.cache
def _make_dispatch():
    mesh = plsc.VectorSubcoreMesh(core_axis_name="c", subcore_axis_name="s")

    @functools.partial(
        pl.kernel, mesh=mesh,
        out_type=jax.ShapeDtypeStruct((DISP_ROWS, D), jnp.float32),
        scratch_types=[
            pltpu.VMEM((_SUB, D), jnp.float32),
            pltpu.VMEM((_SUB,), jnp.int32),
            pltpu.SemaphoreType.DMA,
        ],
    )
    def dispatch(x_hbm, addr_hbm, disp_hbm, rows_v, a_v, sem):
        wid = lax.axis_index("s") * 2 + lax.axis_index("c")
        for sub in range(_TPW // _SUB):
            base = wid * _TPW + sub * _SUB
            xbase = (wid % 16) * _TPW + sub * _SUB
            pltpu.sync_copy(x_hbm.at[pl.ds(xbase, _SUB)], rows_v)
            pltpu.sync_copy(addr_hbm.at[pl.ds(base, _SUB)], a_v)
            copies = []
            for k in range(_SUB // 16):
                idxv = a_v[pl.ds(k * 16, 16)]
                copies.append(pltpu.async_copy(
                    rows_v.at[pl.ds(k * 16, 16)], disp_hbm.at[idxv], sem))
            for cp in copies:
                cp.wait()

    return dispatch


_BH = 768
_NHB = H // _BH


def _ffn_body(disp_ref, w1_ref, b1_ref, w2_ref, b2_ref, y_ref):
    hb = pl.program_id(1)
    d_blk = disp_ref[...].astype(jnp.bfloat16)
    h = lax.dot_general(d_blk, w1_ref[0],
                        (((1,), (1,)), ((), ())),
                        preferred_element_type=jnp.float32)
    h = jnp.maximum(h + b1_ref[0], 0.0)
    acc = jnp.dot(h.astype(jnp.bfloat16), w2_ref[0],
                  preferred_element_type=jnp.float32)

    @pl.when(hb == 0)
    def _():
        y_ref[...] = acc + b2_ref[0]

    @pl.when(hb != 0)
    def _():
        y_ref[...] += acc


def _ffn(disp, fc1_w, fc1_b, fc2_w, fc2_b):
    return pl.pallas_call(
        _ffn_body,
        grid=(E, _NHB),
        in_specs=[
            pl.BlockSpec((CAP, D), lambda e, hb: (e, 0)),
            pl.BlockSpec((1, _BH, D), lambda e, hb: (e, hb, 0)),
            pl.BlockSpec((1, 1, _BH), lambda e, hb: (e, 0, hb)),
            pl.BlockSpec((1, _BH, D), lambda e, hb: (e, hb, 0)),
            pl.BlockSpec((1, 1, D), lambda e, hb: (e, 0, 0)),
        ],
        out_specs=pl.BlockSpec((CAP, D), lambda e, hb: (e, 0)),
        out_shape=jax.ShapeDtypeStruct((E * CAP, D), jnp.float32),
    )(disp, fc1_w.astype(jnp.bfloat16), fc1_b[:, None, :],
      fc2_w.astype(jnp.bfloat16), fc2_b[:, None, :])


@functools.cache
def _make_combine_gather():
    mesh = plsc.VectorSubcoreMesh(core_axis_name="c", subcore_axis_name="s")

    @functools.partial(
        pl.kernel, mesh=mesh,
        out_type=jax.ShapeDtypeStruct((2 * T, D), jnp.float32),
        scratch_types=[
            pltpu.VMEM((_SUB, D), jnp.float32),
            pltpu.VMEM((_SUB,), jnp.int32),
            pltpu.SemaphoreType.DMA,
        ],
    )
    def combine_gather(y_hbm, addr_hbm, out_hbm, rows_v, idx_v, sem):
        wid = lax.axis_index("s") * 2 + lax.axis_index("c")
        for sub in range(_TPW // _SUB):
            base = wid * _TPW + sub * _SUB
            pltpu.sync_copy(addr_hbm.at[pl.ds(base, _SUB)], idx_v)
            pltpu.async_copy(y_hbm.at[idx_v], rows_v, sem).wait()
            pltpu.sync_copy(rows_v, out_hbm.at[pl.ds(base, _SUB)])

    return combine_gather


_BC = 512
_NBC = T // _BC


def _wcombine_body(y1_ref, y2_ref, g1_ref, g2_ref, out_ref):
    out_ref[...] = g1_ref[...] * y1_ref[...] + g2_ref[...] * y2_ref[...]


def _wcombine(y12, g1, g2):
    return pl.pallas_call(
        _wcombine_body,
        grid=(_NBC,),
        in_specs=[
            pl.BlockSpec((_BC, D), lambda j: (j, 0)),
            pl.BlockSpec((_BC, D), lambda j: (j + _NBC, 0)),
            pl.BlockSpec((_BC, 1), lambda j: (j, 0)),
            pl.BlockSpec((_BC, 1), lambda j: (j, 0)),
        ],
        out_specs=pl.BlockSpec((_BC, D), lambda j: (j, 0)),
        out_shape=jax.ShapeDtypeStruct((T, D), jnp.float32),
    )(y12, y12, g1, g2)


def kernel(x, wg, fc1_w, fc1_b, fc2_w, fc2_b):
    a1s, a2s, a1c, a2c, g1, g2 = _gating(x, wg)
    addr_s = jnp.concatenate([a1s[:, 0], a2s[:, 0]])
    addr_c = jnp.concatenate([a1c[:, 0], a2c[:, 0]])
    disp = _make_dispatch()(x, addr_s)
    y = _ffn(disp, fc1_w, fc1_b, fc2_w, fc2_b)
    y12 = _make_combine_gather()(y, addr_c)
    return _wcombine(y12, g1, g2)


# speedup vs baseline: 1.2846x; 1.2846x over previous
import functools

import jax
import jax.numpy as jnp
from jax import lax
from jax.experimental import pallas as pl
from jax.experimental.pallas import tpu as pltpu
from jax.experimental.pallas import tpu_sc as plsc

T, D, H, E = 4096, 768, 3072, 8
CAP = 1280
DUMP = E * CAP
DISP_ROWS = E * CAP + 8

_CH = 512
_NCH = T // _CH


def _gating_body(x_ref, wg_ref, a1s_ref, a2s_ref, a1c_ref, a2c_ref,
                 g1_ref, g2_ref):
    x = x_ref[...]
    wg = wg_ref[...]
    logits = jnp.dot(x, wg, preferred_element_type=jnp.float32)
    m = jnp.max(logits, axis=1, keepdims=True)
    ex = jnp.exp(logits - m)
    s = ex / jnp.sum(ex, axis=1, keepdims=True)

    lane = lax.broadcasted_iota(jnp.int32, (T, E), 1)
    v1 = jnp.max(s, axis=1, keepdims=True)
    i1 = jnp.min(jnp.where(s == v1, lane, E), axis=1, keepdims=True)
    s2 = jnp.where(lane == i1, -jnp.inf, s)
    v2 = jnp.max(s2, axis=1, keepdims=True)
    i2 = jnp.min(jnp.where(s2 == v2, lane, E), axis=1, keepdims=True)

    denom = v1 + v2 + 1e-9
    g1 = v1 / denom
    g2 = v2 / denom

    mask1 = (lane == i1).astype(jnp.float32)
    mask2 = (lane == i2).astype(jnp.float32)
    c1tot = jnp.sum(mask1, axis=0, keepdims=True)

    ri = lax.broadcasted_iota(jnp.int32, (_CH, _CH), 0)
    ci = lax.broadcasted_iota(jnp.int32, (_CH, _CH), 1)
    tril = (ci < ri).astype(jnp.float32)

    c1run = jnp.zeros((1, E), jnp.float32)
    c2run = jnp.zeros((1, E), jnp.float32)
    for j in range(_NCH):
        sl = slice(j * _CH, (j + 1) * _CH)
        m1 = mask1[sl]
        m2 = mask2[sl]
        loc1 = jnp.dot(tril, m1, preferred_element_type=jnp.float32) + c1run
        loc2 = (jnp.dot(tril, m2, preferred_element_type=jnp.float32)
                + c2run + c1tot)
        pos1 = jnp.sum(loc1 * m1, axis=1, keepdims=True)
        pos2 = jnp.sum(loc2 * m2, axis=1, keepdims=True)
        keep1 = pos1 < CAP
        keep2 = pos2 < CAP
        p1 = jnp.minimum(pos1, CAP - 1).astype(jnp.int32)
        p2 = jnp.minimum(pos2, CAP - 1).astype(jnp.int32)
        addr1 = i1[sl] * CAP + p1
        addr2 = i2[sl] * CAP + p2
        a1s_ref[sl, :] = jnp.where(keep1, addr1, DUMP)
        a2s_ref[sl, :] = jnp.where(keep2, addr2, DUMP)
        a1c_ref[sl, :] = addr1
        a2c_ref[sl, :] = addr2
        g1_ref[sl, :] = jnp.where(keep1, g1[sl], 0.0)
        g2_ref[sl, :] = jnp.where(keep2, g2[sl], 0.0)
        c1run = c1run + jnp.sum(m1, axis=0, keepdims=True)
        c2run = c2run + jnp.sum(m2, axis=0, keepdims=True)


def _gating(x, wg):
    i32 = jnp.int32
    f32 = jnp.float32
    return pl.pallas_call(
        _gating_body,
        out_shape=[
            jax.ShapeDtypeStruct((T, 1), i32),
            jax.ShapeDtypeStruct((T, 1), i32),
            jax.ShapeDtypeStruct((T, 1), i32),
            jax.ShapeDtypeStruct((T, 1), i32),
            jax.ShapeDtypeStruct((T, 1), f32),
            jax.ShapeDtypeStruct((T, 1), f32),
        ],
    )(x, wg)


_NW = 32
_TPW = 2 * T // _NW
_SUB = 128


@functools.cache
def _make_dispatch():
    mesh = plsc.VectorSubcoreMesh(core_axis_name="c", subcore_axis_name="s")

    @functools.partial(
        pl.kernel, mesh=mesh,
        out_type=jax.ShapeDtypeStruct((DISP_ROWS, D), jnp.float32),
        scratch_types=[
            pltpu.VMEM((_SUB, D), jnp.float32),
            pltpu.VMEM((_SUB,), jnp.int32),
            pltpu.SemaphoreType.DMA,
        ],
    )
    def dispatch(x_hbm, addr_hbm, disp_hbm, rows_v, a_v, sem):
        wid = lax.axis_index("s") * 2 + lax.axis_index("c")
        for sub in range(_TPW // _SUB):
            base = wid * _TPW + sub * _SUB
            xbase = (wid % 16) * _TPW + sub * _SUB
            pltpu.sync_copy(x_hbm.at[pl.ds(xbase, _SUB)], rows_v)
            pltpu.sync_copy(addr_hbm.at[pl.ds(base, _SUB)], a_v)
            copies = []
            for k in range(_SUB // 16):
                idxv = a_v[pl.ds(k * 16, 16)]
                copies.append(pltpu.async_copy(
                    rows_v.at[pl.ds(k * 16, 16)], disp_hbm.at[idxv], sem))
            for cp in copies:
                cp.wait()

    return dispatch


_BH = 768
_NHB = H // _BH


def _ffn_body(disp_ref, w1_ref, b1_ref, w2_ref, b2_ref, y_ref):
    hb = pl.program_id(1)
    d_blk = disp_ref[...].astype(jnp.bfloat16)
    h = lax.dot_general(d_blk, w1_ref[0].astype(jnp.bfloat16),
                        (((1,), (1,)), ((), ())),
                        preferred_element_type=jnp.float32)
    h = jnp.maximum(h + b1_ref[0], 0.0)
    acc = jnp.dot(h.astype(jnp.bfloat16), w2_ref[0].astype(jnp.bfloat16),
                  preferred_element_type=jnp.float32)

    @pl.when(hb == 0)
    def _():
        y_ref[...] = acc + b2_ref[0]

    @pl.when(hb != 0)
    def _():
        y_ref[...] += acc


def _ffn(disp, fc1_w, fc1_b, fc2_w, fc2_b):
    return pl.pallas_call(
        _ffn_body,
        grid=(E, _NHB),
        in_specs=[
            pl.BlockSpec((CAP, D), lambda e, hb: (e, 0)),
            pl.BlockSpec((1, _BH, D), lambda e, hb: (e, hb, 0)),
            pl.BlockSpec((1, 1, _BH), lambda e, hb: (e, 0, hb)),
            pl.BlockSpec((1, _BH, D), lambda e, hb: (e, hb, 0)),
            pl.BlockSpec((1, 1, D), lambda e, hb: (e, 0, 0)),
        ],
        out_specs=pl.BlockSpec((CAP, D), lambda e, hb: (e, 0)),
        out_shape=jax.ShapeDtypeStruct((E * CAP, D), jnp.float32),
    )(disp, fc1_w, fc1_b[:, None, :], fc2_w, fc2_b[:, None, :])


@functools.cache
def _make_combine_gather():
    mesh = plsc.VectorSubcoreMesh(core_axis_name="c", subcore_axis_name="s")

    @functools.partial(
        pl.kernel, mesh=mesh,
        out_type=jax.ShapeDtypeStruct((2 * T, D), jnp.float32),
        scratch_types=[
            pltpu.VMEM((_SUB, D), jnp.float32),
            pltpu.VMEM((_SUB,), jnp.int32),
            pltpu.SemaphoreType.DMA,
        ],
    )
    def combine_gather(y_hbm, addr_hbm, out_hbm, rows_v, idx_v, sem):
        wid = lax.axis_index("s") * 2 + lax.axis_index("c")
        for sub in range(_TPW // _SUB):
            base = wid * _TPW + sub * _SUB
            pltpu.sync_copy(addr_hbm.at[pl.ds(base, _SUB)], idx_v)
            pltpu.async_copy(y_hbm.at[idx_v], rows_v, sem).wait()
            pltpu.sync_copy(rows_v, out_hbm.at[pl.ds(base, _SUB)])

    return combine_gather


_BC = 512
_NBC = T // _BC


def _wcombine_body(y1_ref, y2_ref, g1_ref, g2_ref, out_ref):
    out_ref[...] = g1_ref[...] * y1_ref[...] + g2_ref[...] * y2_ref[...]


def _wcombine(y12, g1, g2):
    return pl.pallas_call(
        _wcombine_body,
        grid=(_NBC,),
        in_specs=[
            pl.BlockSpec((_BC, D), lambda j: (j, 0)),
            pl.BlockSpec((_BC, D), lambda j: (j + _NBC, 0)),
            pl.BlockSpec((_BC, 1), lambda j: (j, 0)),
            pl.BlockSpec((_BC, 1), lambda j: (j, 0)),
        ],
        out_specs=pl.BlockSpec((_BC, D), lambda j: (j, 0)),
        out_shape=jax.ShapeDtypeStruct((T, D), jnp.float32),
    )(y12, y12, g1, g2)


def kernel(x, wg, fc1_w, fc1_b, fc2_w, fc2_b):
    a1s, a2s, a1c, a2c, g1, g2 = _gating(x, wg)
    addr_s = jnp.concatenate([a1s[:, 0], a2s[:, 0]])
    addr_c = jnp.concatenate([a1c[:, 0], a2c[:, 0]])
    disp = _make_dispatch()(x, addr_s)
    y = _ffn(disp, fc1_w, fc1_b, fc2_w, fc2_b)
    y12 = _make_combine_gather()(y, addr_c)
    return _wcombine(y12, g1, g2)


# speedup vs baseline: 1.3289x; 1.0345x over previous
import functools

import jax
import jax.numpy as jnp
from jax import lax
from jax.experimental import pallas as pl
from jax.experimental.pallas import tpu as pltpu
from jax.experimental.pallas import tpu_sc as plsc

T, D, H, E = 4096, 768, 3072, 8
CAP = 1280
DUMP = E * CAP
DISP_ROWS = E * CAP + 8

_CH = 512
_NCH = T // _CH


def _gating_body(x_ref, wg_ref, as_ref, ac_ref, grep_ref):
    x = x_ref[...]
    wg = wg_ref[...]
    logits = jnp.dot(x, wg, preferred_element_type=jnp.float32)
    m = jnp.max(logits, axis=1, keepdims=True)
    ex = jnp.exp(logits - m)
    s = ex / jnp.sum(ex, axis=1, keepdims=True)

    lane = lax.broadcasted_iota(jnp.int32, (T, E), 1)
    v1 = jnp.max(s, axis=1, keepdims=True)
    i1 = jnp.min(jnp.where(s == v1, lane, E), axis=1, keepdims=True)
    s2 = jnp.where(lane == i1, -jnp.inf, s)
    v2 = jnp.max(s2, axis=1, keepdims=True)
    i2 = jnp.min(jnp.where(s2 == v2, lane, E), axis=1, keepdims=True)

    denom = v1 + v2 + 1e-9
    g1 = v1 / denom
    g2 = v2 / denom

    mask1 = (lane == i1).astype(jnp.float32)
    mask2 = (lane == i2).astype(jnp.float32)
    c1tot = jnp.sum(mask1, axis=0, keepdims=True)

    ri = lax.broadcasted_iota(jnp.int32, (_CH, _CH), 0)
    ci = lax.broadcasted_iota(jnp.int32, (_CH, _CH), 1)
    tril = (ci < ri).astype(jnp.float32)

    c1run = jnp.zeros((1, E), jnp.float32)
    c2run = jnp.zeros((1, E), jnp.float32)
    for j in range(_NCH):
        sl = slice(j * _CH, (j + 1) * _CH)
        sl2 = slice(T + j * _CH, T + (j + 1) * _CH)
        m1 = mask1[sl]
        m2 = mask2[sl]
        loc1 = jnp.dot(tril, m1, preferred_element_type=jnp.float32) + c1run
        loc2 = (jnp.dot(tril, m2, preferred_element_type=jnp.float32)
                + c2run + c1tot)
        pos1 = jnp.sum(loc1 * m1, axis=1, keepdims=True)
        pos2 = jnp.sum(loc2 * m2, axis=1, keepdims=True)
        keep1 = pos1 < CAP
        keep2 = pos2 < CAP
        p1 = jnp.minimum(pos1, CAP - 1).astype(jnp.int32)
        p2 = jnp.minimum(pos2, CAP - 1).astype(jnp.int32)
        addr1 = i1[sl] * CAP + p1
        addr2 = i2[sl] * CAP + p2
        as_ref[sl, :] = jnp.where(keep1, addr1, DUMP)
        as_ref[sl2, :] = jnp.where(keep2, addr2, DUMP)
        ac_ref[sl, :] = addr1
        ac_ref[sl2, :] = addr2
        grep_ref[sl, :] = jnp.broadcast_to(
            jnp.where(keep1, g1[sl], 0.0), (_CH, 128))
        grep_ref[sl2, :] = jnp.broadcast_to(
            jnp.where(keep2, g2[sl], 0.0), (_CH, 128))
        c1run = c1run + jnp.sum(m1, axis=0, keepdims=True)
        c2run = c2run + jnp.sum(m2, axis=0, keepdims=True)


def _gating(x, wg):
    i32 = jnp.int32
    f32 = jnp.float32
    return pl.pallas_call(
        _gating_body,
        out_shape=[
            jax.ShapeDtypeStruct((2 * T, 1), i32),
            jax.ShapeDtypeStruct((2 * T, 1), i32),
            jax.ShapeDtypeStruct((2 * T, 128), f32),
        ],
    )(x, wg)


_NW = 32
_TPW = 2 * T // _NW
_SUB = 128


@functools.cache
def _make_dispatch():
    mesh = plsc.VectorSubcoreMesh(core_axis_name="c", subcore_axis_name="s")

    @functools.partial(
        pl.kernel, mesh=mesh,
        out_type=[
            jax.ShapeDtypeStruct((DISP_ROWS, D), jnp.float32),
            jax.ShapeDtypeStruct((DISP_ROWS, 128), jnp.float32),
        ],
        scratch_types=[
            pltpu.VMEM((_SUB, D), jnp.float32),
            pltpu.VMEM((_SUB, 128), jnp.float32),
            pltpu.VMEM((_SUB,), jnp.int32),
            pltpu.SemaphoreType.DMA,
        ],
    )
    def dispatch(x_hbm, addr_hbm, grep_hbm, disp_hbm, gslot_hbm,
                 rows_v, g_v, a_v, sem):
        wid = lax.axis_index("s") * 2 + lax.axis_index("c")
        for sub in range(_TPW // _SUB):
            base = wid * _TPW + sub * _SUB
            xbase = (wid % 16) * _TPW + sub * _SUB
            pltpu.sync_copy(x_hbm.at[pl.ds(xbase, _SUB)], rows_v)
            pltpu.sync_copy(grep_hbm.at[pl.ds(base, _SUB)], g_v)
            pltpu.sync_copy(addr_hbm.at[pl.ds(base, _SUB)], a_v)
            copies = []
            for k in range(_SUB // 16):
                idxv = a_v[pl.ds(k * 16, 16)]
                copies.append(pltpu.async_copy(
                    rows_v.at[pl.ds(k * 16, 16)], disp_hbm.at[idxv], sem))
                copies.append(pltpu.async_copy(
                    g_v.at[pl.ds(k * 16, 16)], gslot_hbm.at[idxv], sem))
            for cp in copies:
                cp.wait()

    return dispatch


_BH = 768
_NHB = H // _BH


def _ffn_body(disp_ref, g_ref, w1_ref, b1_ref, w2_ref, b2_ref, y_ref):
    hb = pl.program_id(1)
    d_blk = disp_ref[...].astype(jnp.bfloat16)
    h = lax.dot_general(d_blk, w1_ref[0].astype(jnp.bfloat16),
                        (((1,), (1,)), ((), ())),
                        preferred_element_type=jnp.float32)
    h = jnp.maximum(h + b1_ref[0], 0.0)
    acc = jnp.dot(h.astype(jnp.bfloat16), w2_ref[0].astype(jnp.bfloat16),
                  preferred_element_type=jnp.float32)

    @pl.when(hb == 0)
    def _():
        y_ref[...] = acc + b2_ref[0]

    @pl.when((hb != 0) & (hb != _NHB - 1))
    def _():
        y_ref[...] += acc

    @pl.when(hb == _NHB - 1)
    def _():
        y_ref[...] = (y_ref[...] + acc) * g_ref[...][:, 0:1]


def _ffn(disp, gslot, fc1_w, fc1_b, fc2_w, fc2_b):
    return pl.pallas_call(
        _ffn_body,
        grid=(E, _NHB),
        in_specs=[
            pl.BlockSpec((CAP, D), lambda e, hb: (e, 0)),
            pl.BlockSpec((CAP, 128), lambda e, hb: (e, 0)),
            pl.BlockSpec((1, _BH, D), lambda e, hb: (e, hb, 0)),
            pl.BlockSpec((1, 1, _BH), lambda e, hb: (e, 0, hb)),
            pl.BlockSpec((1, _BH, D), lambda e, hb: (e, hb, 0)),
            pl.BlockSpec((1, 1, D), lambda e, hb: (e, 0, 0)),
        ],
        out_specs=pl.BlockSpec((CAP, D), lambda e, hb: (e, 0)),
        out_shape=jax.ShapeDtypeStruct((E * CAP, D), jnp.float32),
    )(disp, gslot, fc1_w, fc1_b[:, None, :], fc2_w, fc2_b[:, None, :])


_CSUB = 64


@functools.cache
def _make_combine():
    mesh = plsc.VectorSubcoreMesh(core_axis_name="c", subcore_axis_name="s")

    @functools.partial(
        pl.kernel, mesh=mesh,
        out_type=jax.ShapeDtypeStruct((T, D), jnp.float32),
        scratch_types=[
            pltpu.VMEM((_CSUB, D), jnp.float32),
            pltpu.VMEM((_CSUB, D), jnp.float32),
            pltpu.VMEM((_CSUB,), jnp.int32),
            pltpu.VMEM((_CSUB,), jnp.int32),
            pltpu.SemaphoreType.DMA,
        ],
    )
    def combine(y_hbm, addr_hbm, out_hbm, r1_v, r2_v, i1_v, i2_v, sem):
        wid = lax.axis_index("s") * 2 + lax.axis_index("c")
        for sub in range(T // _NW // _CSUB):
            base = wid * (T // _NW) + sub * _CSUB
            pltpu.sync_copy(addr_hbm.at[pl.ds(base, _CSUB)], i1_v)
            pltpu.sync_copy(addr_hbm.at[pl.ds(T + base, _CSUB)], i2_v)
            cp1 = pltpu.async_copy(y_hbm.at[i1_v], r1_v, sem)
            cp2 = pltpu.async_copy(y_hbm.at[i2_v], r2_v, sem)
            cp1.wait()
            cp2.wait()

            def row_add(r, carry):
                for cs in range(D // 16):
                    csl = pl.ds(cs * 16, 16)
                    r1_v[r, csl] = r1_v[r, csl] + r2_v[r, csl]
                return carry

            lax.fori_loop(0, _CSUB, row_add, 0)
            pltpu.sync_copy(r1_v, out_hbm.at[pl.ds(base, _CSUB)])

    return combine


def kernel(x, wg, fc1_w, fc1_b, fc2_w, fc2_b):
    addr_s, addr_c, grep = _gating(x, wg)
    disp, gslot = _make_dispatch()(x, addr_s[:, 0], grep)
    y = _ffn(disp, gslot, fc1_w, fc1_b, fc2_w, fc2_b)
    return _make_combine()(y, addr_c[:, 0])


# speedup vs baseline: 1.3866x; 1.0434x over previous
import functools

import jax
import jax.numpy as jnp
from jax import lax
from jax.experimental import pallas as pl
from jax.experimental.pallas import tpu as pltpu
from jax.experimental.pallas import tpu_sc as plsc

T, D, H, E = 4096, 768, 3072, 8
CAP = 1280
DUMP = E * CAP
DISP_ROWS = E * CAP + 8

_CH = 512
_NCH = T // _CH


def _gating_body(x_ref, wg_ref, as_ref, ac_ref, grep_ref):
    x = x_ref[...]
    wg = wg_ref[...]
    logits = jnp.dot(x, wg, preferred_element_type=jnp.float32)
    m = jnp.max(logits, axis=1, keepdims=True)
    ex = jnp.exp(logits - m)
    s = ex / jnp.sum(ex, axis=1, keepdims=True)

    lane = lax.broadcasted_iota(jnp.int32, (T, E), 1)
    v1 = jnp.max(s, axis=1, keepdims=True)
    i1 = jnp.min(jnp.where(s == v1, lane, E), axis=1, keepdims=True)
    s2 = jnp.where(lane == i1, -jnp.inf, s)
    v2 = jnp.max(s2, axis=1, keepdims=True)
    i2 = jnp.min(jnp.where(s2 == v2, lane, E), axis=1, keepdims=True)

    denom = v1 + v2 + 1e-9
    g1 = v1 / denom
    g2 = v2 / denom

    mask1 = (lane == i1).astype(jnp.float32)
    mask2 = (lane == i2).astype(jnp.float32)
    c1tot = jnp.sum(mask1, axis=0, keepdims=True)

    ri = lax.broadcasted_iota(jnp.int32, (_CH, _CH), 0)
    ci = lax.broadcasted_iota(jnp.int32, (_CH, _CH), 1)
    tril = (ci < ri).astype(jnp.float32)

    c1run = jnp.zeros((1, E), jnp.float32)
    c2run = jnp.zeros((1, E), jnp.float32)
    for j in range(_NCH):
        sl = slice(j * _CH, (j + 1) * _CH)
        sl2 = slice(T + j * _CH, T + (j + 1) * _CH)
        m1 = mask1[sl]
        m2 = mask2[sl]
        loc1 = jnp.dot(tril, m1, preferred_element_type=jnp.float32) + c1run
        loc2 = (jnp.dot(tril, m2, preferred_element_type=jnp.float32)
                + c2run + c1tot)
        pos1 = jnp.sum(loc1 * m1, axis=1, keepdims=True)
        pos2 = jnp.sum(loc2 * m2, axis=1, keepdims=True)
        keep1 = pos1 < CAP
        keep2 = pos2 < CAP
        p1 = jnp.minimum(pos1, CAP - 1).astype(jnp.int32)
        p2 = jnp.minimum(pos2, CAP - 1).astype(jnp.int32)
        addr1 = i1[sl] * CAP + p1
        addr2 = i2[sl] * CAP + p2
        as_ref[sl, :] = jnp.where(keep1, addr1, DUMP)
        as_ref[sl2, :] = jnp.where(keep2, addr2, DUMP)
        ac_ref[sl, :] = addr1
        ac_ref[sl2, :] = addr2
        grep_ref[sl, :] = jnp.broadcast_to(
            jnp.where(keep1, g1[sl], 0.0), (_CH, 128))
        grep_ref[sl2, :] = jnp.broadcast_to(
            jnp.where(keep2, g2[sl], 0.0), (_CH, 128))
        c1run = c1run + jnp.sum(m1, axis=0, keepdims=True)
        c2run = c2run + jnp.sum(m2, axis=0, keepdims=True)


def _gating(x, wg):
    i32 = jnp.int32
    f32 = jnp.float32
    return pl.pallas_call(
        _gating_body,
        out_shape=[
            jax.ShapeDtypeStruct((2 * T, 1), i32),
            jax.ShapeDtypeStruct((2 * T, 1), i32),
            jax.ShapeDtypeStruct((2 * T, 128), f32),
        ],
    )(x, wg)


_NW = 32
_TPW = 2 * T // _NW
_SUB = 128


@functools.cache
def _make_dispatch():
    mesh = plsc.VectorSubcoreMesh(core_axis_name="c", subcore_axis_name="s")

    @functools.partial(
        pl.kernel, mesh=mesh,
        out_type=[
            jax.ShapeDtypeStruct((DISP_ROWS, D), jnp.float32),
            jax.ShapeDtypeStruct((DISP_ROWS, 128), jnp.float32),
        ],
        scratch_types=[
            pltpu.VMEM((_SUB, D), jnp.float32),
            pltpu.VMEM((_SUB, 128), jnp.float32),
            pltpu.VMEM((_SUB,), jnp.int32),
            pltpu.SemaphoreType.DMA,
        ],
    )
    def dispatch(x_hbm, addr_hbm, grep_hbm, disp_hbm, gslot_hbm,
                 rows_v, g_v, a_v, sem):
        wid = lax.axis_index("s") * 2 + lax.axis_index("c")
        for sub in range(_TPW // _SUB):
            base = wid * _TPW + sub * _SUB
            xbase = (wid % 16) * _TPW + sub * _SUB
            pltpu.sync_copy(x_hbm.at[pl.ds(xbase, _SUB)], rows_v)
            pltpu.sync_copy(grep_hbm.at[pl.ds(base, _SUB)], g_v)
            pltpu.sync_copy(addr_hbm.at[pl.ds(base, _SUB)], a_v)
            copies = []
            for k in range(_SUB // 16):
                idxv = a_v[pl.ds(k * 16, 16)]
                copies.append(pltpu.async_copy(
                    rows_v.at[pl.ds(k * 16, 16)], disp_hbm.at[idxv], sem))
                copies.append(pltpu.async_copy(
                    g_v.at[pl.ds(k * 16, 16)], gslot_hbm.at[idxv], sem))
            for cp in copies:
                cp.wait()

    return dispatch


_BC2 = 640
_NC2 = CAP // _BC2


def _ffn_body(disp_ref, g_ref, w1_ref, b1_ref, w2_ref, b2_ref, y_ref):
    h = lax.dot_general(disp_ref[...], w1_ref[0],
                        (((1,), (1,)), ((), ())),
                        preferred_element_type=jnp.float32)
    h = jnp.maximum(h + b1_ref[0], 0.0)
    y = jnp.dot(h, w2_ref[0], preferred_element_type=jnp.float32)
    y_ref[...] = (y + b2_ref[0]) * g_ref[...][:, 0:1]


def _ffn(disp, gslot, fc1_w, fc1_b, fc2_w, fc2_b):
    return pl.pallas_call(
        _ffn_body,
        grid=(E, _NC2),
        in_specs=[
            pl.BlockSpec((_BC2, D), lambda e, c: (e * _NC2 + c, 0)),
            pl.BlockSpec((_BC2, 128), lambda e, c: (e * _NC2 + c, 0)),
            pl.BlockSpec((1, H, D), lambda e, c: (e, 0, 0)),
            pl.BlockSpec((1, 1, H), lambda e, c: (e, 0, 0)),
            pl.BlockSpec((1, H, D), lambda e, c: (e, 0, 0)),
            pl.BlockSpec((1, 1, D), lambda e, c: (e, 0, 0)),
        ],
        out_specs=pl.BlockSpec((_BC2, D), lambda e, c: (e * _NC2 + c, 0)),
        out_shape=jax.ShapeDtypeStruct((E * CAP, D), jnp.float32),
    )(disp, gslot, fc1_w, fc1_b[:, None, :], fc2_w, fc2_b[:, None, :])


_CSUB = 64


@functools.cache
def _make_combine():
    mesh = plsc.VectorSubcoreMesh(core_axis_name="c", subcore_axis_name="s")

    @functools.partial(
        pl.kernel, mesh=mesh,
        out_type=jax.ShapeDtypeStruct((T, D), jnp.float32),
        scratch_types=[
            pltpu.VMEM((_CSUB, D), jnp.float32),
            pltpu.VMEM((_CSUB, D), jnp.float32),
            pltpu.VMEM((_CSUB,), jnp.int32),
            pltpu.VMEM((_CSUB,), jnp.int32),
            pltpu.SemaphoreType.DMA,
        ],
    )
    def combine(y_hbm, addr_hbm, out_hbm, r1_v, r2_v, i1_v, i2_v, sem):
        wid = lax.axis_index("s") * 2 + lax.axis_index("c")
        for sub in range(T // _NW // _CSUB):
            base = wid * (T // _NW) + sub * _CSUB
            pltpu.sync_copy(addr_hbm.at[pl.ds(base, _CSUB)], i1_v)
            pltpu.sync_copy(addr_hbm.at[pl.ds(T + base, _CSUB)], i2_v)
            cp1 = pltpu.async_copy(y_hbm.at[i1_v], r1_v, sem)
            cp2 = pltpu.async_copy(y_hbm.at[i2_v], r2_v, sem)
            cp1.wait()
            cp2.wait()

            def row_add(r, carry):
                for cs in range(D // 16):
                    csl = pl.ds(cs * 16, 16)
                    r1_v[r, csl] = r1_v[r, csl] + r2_v[r, csl]
                return carry

            lax.fori_loop(0, _CSUB, row_add, 0)
            pltpu.sync_copy(r1_v, out_hbm.at[pl.ds(base, _CSUB)])

    return combine


def kernel(x, wg, fc1_w, fc1_b, fc2_w, fc2_b):
    addr_s, addr_c, grep = _gating(x, wg)
    disp, gslot = _make_dispatch()(x, addr_s[:, 0], grep)
    y = _ffn(disp, gslot, fc1_w, fc1_b, fc2_w, fc2_b)
    return _make_combine()(y, addr_c[:, 0])
